# baseline (speedup 1.0000x reference)
import jax
import jax.numpy as jnp
from jax import lax
from jax.experimental import pallas as pl
from jax.experimental.pallas import tpu as pltpu
from jax.experimental.pallas import tpu_sc as plsc

NUM_HID = 128
HALF = NUM_HID // 2
SEQ = 200
CHUNK = 64
LANES = 16
NBUF = 5
AHEAD = 3
NW = 32
SEQ_PER_W = 128
JBLK = SEQ_PER_W // CHUNK


def _emb_body(x_hbm, emb_hbm, pos_hbm, out_hbm, idx_all, pos_v, pat_v,
              *scr):
    rows = scr[:NBUF]
    outs = scr[NBUF:2 * NBUF]
    sin = scr[2 * NBUF:3 * NBUF]
    sout = scr[3 * NBUF:4 * NBUF]
    oidx = scr[4 * NBUF:5 * NBUF]
    wid = lax.axis_index("s") * 2 + lax.axis_index("c")
    nch = x_hbm.shape[0] // NW
    wbase = wid * nch * CHUNK

    pltpu.sync_copy(x_hbm.at[pl.ds(wid * nch, nch)], idx_all)
    pltpu.sync_copy(pos_hbm, pos_v)

    for i in range(CHUNK // LANES):
        pat_v[pl.ds(i * LANES, LANES)] = (
            lax.iota(jnp.int32, LANES) + i * LANES) * SEQ

    def start_gather(g, b):
        pltpu.async_copy(emb_hbm.at[idx_all.at[g]], rows[b], sin[b])

    def wait_gather(b):
        pltpu.make_async_copy(emb_hbm.at[pl.ds(0, CHUNK)], rows[b],
                              sin[b]).wait()

    def start_store(g, b):
        j = g // SEQ
        p = g % SEQ
        cbase = wbase + j * CHUNK * SEQ + p
        for i in range(CHUNK // LANES):
            oidx[b][pl.ds(i * LANES, LANES)] = (
                pat_v[pl.ds(i * LANES, LANES)] + cbase)
        pltpu.async_copy(outs[b], out_hbm.at[oidx[b]], sout[b])

    def wait_store(b):
        pltpu.make_async_copy(outs[b], out_hbm.at[oidx[b]], sout[b]).wait()

    for j in range(AHEAD):
        start_gather(j, j)

    def outer(t, carry):
        for b in range(NBUF):
            g = t * NBUF + b
            bb = (b + AHEAD) % NBUF

            @pl.when(g < nch - AHEAD)
            def _():
                start_gather(g + AHEAD, bb)

            @pl.when(g >= NBUF)
            def _():
                wait_store(b)

            wait_gather(b)
            p = g % SEQ
            vs = [pos_v[p, pl.ds(j * LANES, LANES)]
                  for j in range(NUM_HID // LANES)]

            @plsc.parallel_loop(0, CHUNK, unroll=2)
            def row_body(r):
                for j in range(HALF // LANES):
                    w = rows[b][r, pl.ds(j * LANES, LANES)]
                    bf = plsc.bitcast(w, jnp.bfloat16)
                    lo, hi = plsc.unpack(bf, format=plsc.PackFormat.INTERLEAVED)
                    outs[b][r, pl.ds(2 * j * LANES, LANES)] = lo + vs[2 * j]
                    outs[b][r, pl.ds((2 * j + 1) * LANES, LANES)] = (
                        hi + vs[2 * j + 1])

            start_store(g, b)
        return carry

    lax.fori_loop(0, nch // NBUF, outer, 0)

    for b in range(NBUF):
        wait_store(b)


def kernel(x, emb, pos_emb):
    bsz, s = x.shape
    h = emb.shape[1]
    xi = x.astype(jnp.int32).reshape(NW, JBLK, CHUNK, s)
    xi = xi.transpose(0, 1, 3, 2).reshape(bsz * s // CHUNK, CHUNK)
    v = emb.shape[0]
    emb_pk = lax.bitcast_convert_type(
        emb.astype(jnp.bfloat16).reshape(v, h // 32, 2, LANES)
        .transpose(0, 1, 3, 2),
        jnp.int32).reshape(v, HALF)

    mesh = plsc.VectorSubcoreMesh(core_axis_name="c", subcore_axis_name="s")
    run = pl.kernel(
        _emb_body,
        out_type=jax.ShapeDtypeStruct((bsz * s, h), jnp.float32),
        mesh=mesh,
        compiler_params=pltpu.CompilerParams(use_tc_tiling_on_sc=False,
                                             needs_layout_passes=False),
        scratch_types=(
            [pltpu.VMEM(((bsz * s) // (NW * CHUNK), CHUNK), jnp.int32),
             pltpu.VMEM((s, h), jnp.float32),
             pltpu.VMEM((CHUNK,), jnp.int32)]
            + [pltpu.VMEM((CHUNK, HALF), jnp.int32)] * NBUF
            + [pltpu.VMEM((CHUNK, h), jnp.float32)] * NBUF
            + [pltpu.SemaphoreType.DMA] * (2 * NBUF)
            + [pltpu.VMEM((CHUNK,), jnp.int32)] * NBUF
        ),
    )
    out = run(xi, emb_pk, pos_emb)
    return out.reshape(bsz, s, h)


# speedup vs baseline: 1.3844x; 1.3844x over previous
import jax
import jax.numpy as jnp
from jax import lax
from jax.experimental import pallas as pl
from jax.experimental.pallas import tpu as pltpu
from jax.experimental.pallas import tpu_sc as plsc

NUM_HID = 128
SEQ = 200
CHUNK = 128
LANES = 16
NBUF = 4
AHEAD = 2
NW = 32
SEQ_PER_W = 128
JBLK = SEQ_PER_W // CHUNK


def _emb_body(x_hbm, emb_hbm, pos_hbm, out_hbm, idx_all, pos_v, pat_v,
              *scr):
    rows = scr[:NBUF]
    sin = scr[NBUF:2 * NBUF]
    sout = scr[2 * NBUF:3 * NBUF]
    sin2 = scr[4 * NBUF:5 * NBUF]
    oidx = scr[3 * NBUF:4 * NBUF]
    wid = lax.axis_index("s") * 2 + lax.axis_index("c")
    nch = x_hbm.shape[0] // NW
    wbase = wid * nch * CHUNK

    pltpu.sync_copy(x_hbm.at[pl.ds(wid * nch, nch)], idx_all)
    pltpu.sync_copy(pos_hbm, pos_v)

    for i in range(CHUNK // LANES):
        pat_v[pl.ds(i * LANES, LANES)] = (
            lax.iota(jnp.int32, LANES) + i * LANES) * SEQ

    HC = CHUNK // 2

    def start_gather(g, b):
        pltpu.async_copy(emb_hbm.at[idx_all.at[g, pl.ds(0, HC)]],
                         rows[b].at[pl.ds(0, HC)], sin[b])
        pltpu.async_copy(emb_hbm.at[idx_all.at[g, pl.ds(HC, HC)]],
                         rows[b].at[pl.ds(HC, HC)], sin2[b])

    def wait_gather(b):
        pltpu.make_async_copy(emb_hbm.at[pl.ds(0, HC)],
                              rows[b].at[pl.ds(0, HC)], sin[b]).wait()
        pltpu.make_async_copy(emb_hbm.at[pl.ds(0, HC)],
                              rows[b].at[pl.ds(HC, HC)], sin2[b]).wait()

    def start_store(g, b):
        j = g // SEQ
        p = g % SEQ
        cbase = wbase + j * CHUNK * SEQ + p
        for i in range(CHUNK // LANES):
            oidx[b][pl.ds(i * LANES, LANES)] = (
                pat_v[pl.ds(i * LANES, LANES)] + cbase)
        pltpu.async_copy(rows[b], out_hbm.at[oidx[b]], sout[b])

    def wait_store(b):
        pltpu.make_async_copy(rows[b], out_hbm.at[oidx[b]], sout[b]).wait()

    for j in range(AHEAD):
        start_gather(j, j)

    def outer(t, carry):
        for b in range(NBUF):
            g = t * NBUF + b
            bb = (b + AHEAD) % NBUF

            @pl.when(jnp.logical_and(g >= NBUF - AHEAD, g < nch - AHEAD))
            def _():
                wait_store(bb)

            @pl.when(g < nch - AHEAD)
            def _():
                start_gather(g + AHEAD, bb)

            wait_gather(b)
            p = g % SEQ
            vs = [pos_v[p, pl.ds(j * LANES, LANES)]
                  for j in range(NUM_HID // LANES)]

            @plsc.parallel_loop(0, CHUNK, unroll=2)
            def row_body(r):
                for j in range(NUM_HID // LANES):
                    plsc.addupdate(rows[b].at[r, pl.ds(j * LANES, LANES)],
                                   vs[j])

            start_store(g, b)
        return carry

    lax.fori_loop(0, nch // NBUF, outer, 0)

    for b in range(NBUF):
        wait_store(b)


def kernel(x, emb, pos_emb):
    bsz, s = x.shape
    h = emb.shape[1]
    xi = x.astype(jnp.int32).reshape(NW, JBLK, CHUNK, s)
    xi = xi.transpose(0, 1, 3, 2).reshape(bsz * s // CHUNK, CHUNK)

    mesh = plsc.VectorSubcoreMesh(core_axis_name="c", subcore_axis_name="s")
    run = pl.kernel(
        _emb_body,
        out_type=jax.ShapeDtypeStruct((bsz * s, h), jnp.float32),
        mesh=mesh,
        scratch_types=(
            [pltpu.VMEM(((bsz * s) // (NW * CHUNK), CHUNK), jnp.int32),
             pltpu.VMEM((s, h), jnp.float32),
             pltpu.VMEM((CHUNK,), jnp.int32)]
            + [pltpu.VMEM((CHUNK, h), jnp.float32)] * NBUF
            + [pltpu.SemaphoreType.DMA] * (2 * NBUF)
            + [pltpu.VMEM((CHUNK,), jnp.int32)] * NBUF
            + [pltpu.SemaphoreType.DMA] * NBUF
        ),
    )
    out = run(xi, emb, pos_emb)
    return out.reshape(bsz, s, h)


# speedup vs baseline: 1.3862x; 1.0013x over previous
import jax
import jax.numpy as jnp
from jax import lax
from jax.experimental import pallas as pl
from jax.experimental.pallas import tpu as pltpu
from jax.experimental.pallas import tpu_sc as plsc

NUM_HID = 128
SEQ = 200
CHUNK = 128
LANES = 16
NBUF = 4
AHEAD = 2
NW = 32
SEQ_PER_W = 128
JBLK = SEQ_PER_W // CHUNK


def _emb_body(x_hbm, emb_hbm, pos_hbm, out_hbm, idx_all, pos_v, pat_v,
              *scr):
    rows = scr[:NBUF]
    sin = scr[NBUF:2 * NBUF]
    sout = scr[2 * NBUF:3 * NBUF]
    oidx = scr[3 * NBUF:4 * NBUF]
    wid = lax.axis_index("s") * 2 + lax.axis_index("c")
    nch = x_hbm.shape[0] // NW
    wbase = wid * nch * CHUNK

    pltpu.sync_copy(x_hbm.at[pl.ds(wid * nch, nch)], idx_all)
    pltpu.sync_copy(pos_hbm, pos_v)

    for i in range(CHUNK // LANES):
        pat_v[pl.ds(i * LANES, LANES)] = (
            lax.iota(jnp.int32, LANES) + i * LANES) * SEQ

    def start_gather(g, b):
        pltpu.async_copy(emb_hbm.at[idx_all.at[g]], rows[b], sin[b])

    def wait_gather(b):
        pltpu.make_async_copy(emb_hbm.at[pl.ds(0, CHUNK)], rows[b],
                              sin[b]).wait()

    def start_store(g, b):
        j = g // SEQ
        p = g % SEQ
        cbase = wbase + j * CHUNK * SEQ + p
        for i in range(CHUNK // LANES):
            oidx[b][pl.ds(i * LANES, LANES)] = (
                pat_v[pl.ds(i * LANES, LANES)] + cbase)
        pltpu.async_copy(rows[b], out_hbm.at[oidx[b]], sout[b])

    def wait_store(b):
        pltpu.make_async_copy(rows[b], out_hbm.at[oidx[b]], sout[b]).wait()

    for j in range(AHEAD):
        start_gather(j, j)

    def outer(t, carry):
        for b in range(NBUF):
            g = t * NBUF + b
            bb = (b + AHEAD) % NBUF

            @pl.when(jnp.logical_and(g >= NBUF - AHEAD, g < nch - AHEAD))
            def _():
                wait_store(bb)

            @pl.when(g < nch - AHEAD)
            def _():
                start_gather(g + AHEAD, bb)

            wait_gather(b)
            p = g % SEQ
            vs = [pos_v[p, pl.ds(j * LANES, LANES)]
                  for j in range(NUM_HID // LANES)]

            @plsc.parallel_loop(0, CHUNK, unroll=2)
            def row_body(r):
                for j in range(NUM_HID // LANES):
                    plsc.addupdate(rows[b].at[r, pl.ds(j * LANES, LANES)],
                                   vs[j])

            start_store(g, b)
        return carry

    lax.fori_loop(0, nch // NBUF, outer, 0)

    for b in range(NBUF):
        wait_store(b)


def kernel(x, emb, pos_emb):
    bsz, s = x.shape
    h = emb.shape[1]
    xi = x.astype(jnp.int32).reshape(NW, JBLK, CHUNK, s)
    xi = xi.transpose(0, 1, 3, 2).reshape(bsz * s // CHUNK, CHUNK)

    mesh = plsc.VectorSubcoreMesh(core_axis_name="c", subcore_axis_name="s")
    run = pl.kernel(
        _emb_body,
        out_type=jax.ShapeDtypeStruct((bsz * s, h), jnp.float32),
        mesh=mesh,
        scratch_types=(
            [pltpu.VMEM(((bsz * s) // (NW * CHUNK), CHUNK), jnp.int32),
             pltpu.VMEM((s, h), jnp.float32),
             pltpu.VMEM((CHUNK,), jnp.int32)]
            + [pltpu.VMEM((CHUNK, h), jnp.float32)] * NBUF
            + [pltpu.SemaphoreType.DMA] * (2 * NBUF)
            + [pltpu.VMEM((CHUNK,), jnp.int32)] * NBUF
        ),
    )
    out = run(xi, emb, pos_emb)
    return out.reshape(bsz, s, h)
